# baseline (speedup 1.0000x reference)
import functools

import numpy as np
import jax
import jax.numpy as jnp
from jax import lax
from jax.experimental import pallas as pl
from jax.experimental.pallas import tpu as pltpu
from jax.experimental.pallas import tpu_sc as plsc

_G = 76
_GG = _G * _G
_NUM = 3
_CP = 14
_NCH = _NUM * _CP
_NCLS = 7
_NW = 32
_NC = 2
_CHUNK = 304
_NCHK = _GG // _CHUNK
_GRP = _CHUNK // 16

_PP = np.arange(_GG, dtype=np.int64)
_GXT = (_PP % _G).astype(np.float32)
_GYT = (_PP // _G).astype(np.float32)
_QT = np.arange(_CHUNK, dtype=np.int32)
_FST = np.repeat(np.arange(15, dtype=np.int32), 16)


def _sig(v):
    return 1.0 / (1.0 + jnp.exp(-v))


def _atan(z):
    az = jnp.abs(z)
    inv = az > 1.0
    u = jnp.where(inv, 1.0 / az, az)
    u2 = u * u
    p = u * (0.9998660 + u2 * (-0.3302995 + u2 * (
        0.1801410 + u2 * (-0.0851330 + u2 * 0.0208351))))
    r = jnp.where(inv, (jnp.pi / 2.0) - p, p)
    return jnp.sign(z) * r


def _make_sc(B):
    n_items = B * _NCHK
    n_iter = (n_items + _NW - 1) // _NW
    mesh = plsc.VectorSubcoreMesh(core_axis_name="c", subcore_axis_name="s")

    @functools.partial(
        pl.kernel,
        out_type=jax.ShapeDtypeStruct((B, _NUM * _GG, 15), jnp.float32),
        mesh=mesh,
        compiler_params=pltpu.CompilerParams(needs_layout_passes=False),
        scratch_types=[
            pltpu.VMEM((_NCH * _CHUNK,), jnp.float32),
            pltpu.VMEM((_NUM * _CHUNK, 15), jnp.float32),
            pltpu.VMEM((_CHUNK,), jnp.float32),
            pltpu.VMEM((_CHUNK,), jnp.float32),
            pltpu.VMEM((_NUM * 32,), jnp.float32),
            pltpu.VMEM((_CHUNK,), jnp.int32),
            pltpu.VMEM((15 * 16,), jnp.int32),
            pltpu.SemaphoreType.DMA,
            pltpu.SemaphoreType.DMA,
        ],
    )
    def sck(x_hbm, gx_hbm, gy_hbm, asp_hbm, q_hbm, fs_hbm, out_hbm,
            slab, stage, gxv, gyv, aspv, qv, fsv, sem_in, sem_out):
        wid = lax.axis_index("s") * _NC + lax.axis_index("c")
        pltpu.sync_copy(asp_hbm, aspv)
        pltpu.sync_copy(q_hbm, qv)
        pltpu.sync_copy(fs_hbm, fsv)
        for it in range(n_iter):
            item = wid + _NW * it

            @pl.when(item < n_items)
            def _process(item=item):
                b = item // _NCHK
                ch = item - b * _NCHK
                start = ch * _CHUNK
                copies = [
                    pltpu.async_copy(gx_hbm.at[pl.ds(start, _CHUNK)],
                                     gxv, sem_in),
                    pltpu.async_copy(gy_hbm.at[pl.ds(start, _CHUNK)],
                                     gyv, sem_in),
                ]
                for c in range(_NCH):
                    copies.append(pltpu.async_copy(
                        x_hbm.at[pl.ds((b * _NCH + c) * _GG + start, _CHUNK)],
                        slab.at[pl.ds(c * _CHUNK, _CHUNK)],
                        sem_in))
                for cp in copies:
                    cp.wait()
                for a in range(_NUM):
                    c0 = a * _CP
                    crows = [((3 * k + a) // _NCLS) * _CP + _NCLS
                             + (3 * k + a) % _NCLS for k in range(_NCLS)]

                    @pl.loop(0, _GRP)
                    def _grp(g, a=a, c0=c0, crows=crows):
                        def row(c):
                            return slab[pl.ds(c * _CHUNK + g * 16, 16)]

                        sl = pl.ds(g * 16, 16)
                        awv = aspv[pl.ds(a * 32, 16)]
                        ahv = aspv[pl.ds(a * 32 + 16, 16)]
                        gx = gxv[sl]
                        gy = gyv[sl]
                        im = row(c0 + 4)
                        re_ = row(c0 + 5)
                        yaw = _atan(im / re_)
                        conf = _sig(row(c0 + 6))
                        ax = ((_sig(row(c0 + 0)) + gx)
                              * 8.0).astype(jnp.int32).astype(jnp.float32)
                        ay = ((_sig(row(c0 + 1)) + gy)
                              * 8.0).astype(jnp.int32).astype(jnp.float32)
                        aw = jnp.exp(row(c0 + 2)) * awv
                        ah = jnp.exp(row(c0 + 3)) * ahv
                        vals = [im, re_, yaw, conf, ax, ay, aw, ah]
                        for k in range(_NCLS):
                            vals.append(row(crows[k]))
                        rowidx = qv[sl] + (a * _CHUNK)
                        for fidx, v in enumerate(vals):
                            plsc.store_scatter(
                                stage,
                                [rowidx, fsv[pl.ds(fidx * 16, 16)]], v)
                outs = []
                for a in range(_NUM):
                    outs.append(pltpu.async_copy(
                        stage.at[pl.ds(a * _CHUNK, _CHUNK), :],
                        out_hbm.at[b, pl.ds(a * _GG + start, _CHUNK), :],
                        sem_out))
                for cp in outs:
                    cp.wait()

    return sck


def kernel(x, anchors):
    B = x.shape[0]
    xr = x.reshape(-1)
    asp = jnp.repeat(anchors.reshape(_NUM * 2), 16).reshape(_NUM * 32)
    out = _make_sc(B)(
        xr, jnp.asarray(_GXT), jnp.asarray(_GYT), asp,
        jnp.asarray(_QT), jnp.asarray(_FST))
    return out


# speedup vs baseline: 1.0629x; 1.0629x over previous
import functools

import jax
import jax.numpy as jnp
from jax import lax
from jax.experimental import pallas as pl
from jax.experimental.pallas import tpu as pltpu
from jax.experimental.pallas import tpu_sc as plsc

_G = 76
_GG = _G * _G
_NUM = 3
_CP = 14
_NCLS = 7
_STRIDE = 8.0
_NW = 32
_NC = 2
_NSLOT = 16
_ROWS = 8
_BOX = _ROWS * _G
_OFFS = (0, 16, 32, 48, 64)


def _sigmoid(v):
    return 1.0 / (1.0 + jnp.exp(-v))


def _arctan(z):
    az = jnp.abs(z)
    inv = az > 1.0
    u = jnp.where(inv, 1.0 / az, az)
    u2 = u * u
    p = u * (0.9998660 + u2 * (-0.3302995 + u2 * (
        0.1801410 + u2 * (-0.0851330 + u2 * 0.0208351))))
    r = jnp.where(inv, (jnp.pi / 2.0) - p, p)
    return jnp.sign(z) * r


def _fields_body(anchors_ref, x_ref, out_ref):
    gx = jax.lax.broadcasted_iota(jnp.int32, (_G, _G), 1).astype(jnp.float32)
    gy = jax.lax.broadcasted_iota(jnp.int32, (_G, _G), 0).astype(jnp.float32)
    for a in range(_NUM):
        c0 = a * _CP
        im = x_ref[0, c0 + 4]
        re_ = x_ref[0, c0 + 5]
        planes = [
            im,
            re_,
            _arctan(im / re_),
            _sigmoid(x_ref[0, c0 + 6]),
            jnp.floor((_sigmoid(x_ref[0, c0 + 0]) + gx) * _STRIDE),
            jnp.floor((_sigmoid(x_ref[0, c0 + 1]) + gy) * _STRIDE),
            jnp.exp(x_ref[0, c0 + 2]) * anchors_ref[a, 0],
            jnp.exp(x_ref[0, c0 + 3]) * anchors_ref[a, 1],
        ]
        for k in range(_NCLS):
            m = 3 * k + a
            planes.append(x_ref[0, (m // _NCLS) * _CP + _NCLS + m % _NCLS])
        for f, pln in enumerate(planes):
            out_ref[0, a * _NSLOT + f, :, 0:_G] = pln


def _fields(x, anchors):
    B = x.shape[0]
    return pl.pallas_call(
        _fields_body,
        grid=(B,),
        in_specs=[
            pl.BlockSpec(memory_space=pltpu.SMEM),
            pl.BlockSpec((1, _NUM * _CP, _G, _G), lambda b: (b, 0, 0, 0)),
        ],
        out_specs=pl.BlockSpec((1, _NUM * _NSLOT, _G, 128),
                               lambda b: (b, 0, 0, 0)),
        out_shape=jax.ShapeDtypeStruct((B, _NUM * _NSLOT, _G, 128),
                                       jnp.float32),
    )(anchors, x)


def _make_interleave(B):
    n_chunk = _G // _ROWS + 1
    n_items = B * n_chunk
    n_iter = (n_items + _NW - 1) // _NW
    mesh = plsc.VectorSubcoreMesh(core_axis_name="c", subcore_axis_name="s")

    @functools.partial(
        pl.kernel,
        out_type=jax.ShapeDtypeStruct((B * _NUM * _GG * 15,), jnp.float32),
        mesh=mesh,
        compiler_params=pltpu.CompilerParams(needs_layout_passes=False),
        scratch_types=[
            pltpu.VMEM((_NUM * _NSLOT, _ROWS, 128), jnp.float32),
            pltpu.VMEM((_NUM * _BOX * 15,), jnp.float32),
            pltpu.SemaphoreType.DMA,
            pltpu.SemaphoreType.DMA,
        ],
    )
    def sck(f_hbm, out_hbm, slab, stage, sem_in, sem_out):
        wid = lax.axis_index("s") * _NC + lax.axis_index("c")
        for it in range(n_iter):
            item = wid + _NW * it

            @pl.when(item < n_items)
            def _process(item=item):
                b = item // n_chunk
                ci = item - b * n_chunk
                tail = ci == (n_chunk - 1)
                i0 = ci * _ROWS
                nrows = jnp.where(tail, _G - (n_chunk - 1) * _ROWS, _ROWS)

                @pl.when(jnp.logical_not(tail))
                def _dma_full():
                    pltpu.async_copy(
                        f_hbm.at[b, :, pl.ds(i0, _ROWS), :],
                        slab, sem_in).wait()

                @pl.when(tail)
                def _dma_tail():
                    pltpu.async_copy(
                        f_hbm.at[b, :, pl.ds(i0, 4), :],
                        slab.at[:, pl.ds(0, 4), :], sem_in).wait()

                for a in range(_NUM):

                    @pl.loop(0, nrows)
                    def _row(r, a=a):
                        @pl.loop(0, 4)
                        def _full(j, a=a, r=r):
                            iota = lax.iota(jnp.int32, 16)
                            o = j * 16
                            base = (r * _G) * 15 + o * 15 + a * (_BOX * 15)
                            idx = iota * 15 + base
                            for f in range(15):
                                v = slab[a * _NSLOT + f, r, pl.ds(o, 16)]
                                plsc.store_scatter(stage, [idx + f], v)

                        @pl.loop(0, 1)
                        def _last(_z, a=a, r=r):
                            iota = lax.iota(jnp.int32, 16)
                            mask = iota < (_G - 64)
                            base = (r * _G + 64) * 15 + a * (_BOX * 15)
                            idx = iota * 15 + base
                            for f in range(15):
                                v = slab[a * _NSLOT + f, r, pl.ds(64, 16)]
                                plsc.store_scatter(stage, [idx + f], v,
                                                   mask=mask)

                obase = b * (_NUM * _GG * 15)

                @pl.when(jnp.logical_not(tail))
                def _out_full():
                    outs = []
                    for a in range(_NUM):
                        outs.append(pltpu.async_copy(
                            stage.at[pl.ds(a * _BOX * 15, _BOX * 15)],
                            out_hbm.at[pl.ds(
                                obase + a * (_GG * 15) + i0 * (_G * 15),
                                _BOX * 15)],
                            sem_out))
                    for cp in outs:
                        cp.wait()

                @pl.when(tail)
                def _out_tail():
                    outs = []
                    for a in range(_NUM):
                        outs.append(pltpu.async_copy(
                            stage.at[pl.ds(a * _BOX * 15, 4 * _G * 15)],
                            out_hbm.at[pl.ds(
                                obase + a * (_GG * 15) + i0 * (_G * 15),
                                4 * _G * 15)],
                            sem_out))
                    for cp in outs:
                        cp.wait()

    return sck


def kernel(x, anchors):
    B = x.shape[0]
    f = _fields(x, anchors)
    out = _make_interleave(B)(f)
    return out.reshape(B, _NUM * _GG, 15)


# speedup vs baseline: 2.8932x; 2.7221x over previous
import jax
import jax.numpy as jnp
from jax.experimental import pallas as pl
from jax.experimental.pallas import tpu as pltpu

_G = 76
_GG = _G * _G
_NUM = 3
_CP = 14
_NCLS = 7
_STRIDE = 8.0


def _sigmoid(v):
    return 1.0 / (1.0 + jnp.exp(-v))


def _arctan(z):
    az = jnp.abs(z)
    inv = az > 1.0
    u = jnp.where(inv, 1.0 / az, az)
    u2 = u * u
    p = u * (0.9998660 + u2 * (-0.3302995 + u2 * (
        0.1801410 + u2 * (-0.0851330 + u2 * 0.0208351))))
    r = jnp.where(inv, (jnp.pi / 2.0) - p, p)
    return jnp.sign(z) * r


def _fields_body(anchors_ref, x_ref, out_ref):
    gx = jax.lax.broadcasted_iota(jnp.int32, (_G, _G), 1).astype(jnp.float32)
    gy = jax.lax.broadcasted_iota(jnp.int32, (_G, _G), 0).astype(jnp.float32)
    for a in range(_NUM):
        c0 = a * _CP
        im = x_ref[0, c0 + 4]
        re_ = x_ref[0, c0 + 5]
        planes = [
            im,
            re_,
            _arctan(im / re_),
            _sigmoid(x_ref[0, c0 + 6]),
            jnp.floor((_sigmoid(x_ref[0, c0 + 0]) + gx) * _STRIDE),
            jnp.floor((_sigmoid(x_ref[0, c0 + 1]) + gy) * _STRIDE),
            jnp.exp(x_ref[0, c0 + 2]) * anchors_ref[a, 0],
            jnp.exp(x_ref[0, c0 + 3]) * anchors_ref[a, 1],
        ]
        for k in range(_NCLS):
            m = 3 * k + a
            planes.append(x_ref[0, (m // _NCLS) * _CP + _NCLS + m % _NCLS])
        for f, pln in enumerate(planes):
            out_ref[0, f, a] = pln


def kernel(x, anchors):
    B = x.shape[0]
    fields = pl.pallas_call(
        _fields_body,
        grid=(B,),
        in_specs=[
            pl.BlockSpec(memory_space=pltpu.SMEM),
            pl.BlockSpec((1, _NUM * _CP, _G, _G), lambda b: (b, 0, 0, 0)),
        ],
        out_specs=pl.BlockSpec((1, 15, _NUM, _G, _G),
                               lambda b: (b, 0, 0, 0, 0)),
        out_shape=jax.ShapeDtypeStruct((B, 15, _NUM, _G, _G), jnp.float32),
    )(anchors, x)
    return fields.transpose(0, 2, 3, 4, 1).reshape(B, _NUM * _GG, 15)
